# initial kernel scaffold (unmeasured)
import jax
import jax.numpy as jnp
from jax import lax
from jax.experimental import pallas as pl
from jax.experimental.pallas import tpu as pltpu

N_Z = 2
T_TILE = 256


def kernel(x, router, W1, W2):
    t_loc, d = x.shape
    e_loc, _, f = W1.shape
    t = N_Z * t_loc
    n_tiles = t // T_TILE

    def body(x_ref, r_ref, w1_any, w2_any, out_ref,
             xfull_ref, rin_ref, wloc_ref,
             w1f_ref, w2f_ref, w1b_ref, w2b_ref,
             acc_ref, sendbuf_ref, comb_ref,
             sem_xs, sem_xr, sem_rs, sem_rr, sem_cs, sem_cr,
             sem_w1, sem_w2):
        my_x = lax.axis_index("x")
        my_y = lax.axis_index("y")
        my_z = lax.axis_index("z")
        peer = (my_x, my_y, 1 - my_z)

        barrier_sem = pltpu.get_barrier_semaphore()
        pl.semaphore_signal(barrier_sem, inc=1, device_id=peer,
                            device_id_type=pl.DeviceIdType.MESH)
        pl.semaphore_wait(barrier_sem, 1)

        rdma_x = pltpu.make_async_remote_copy(
            src_ref=x_ref,
            dst_ref=xfull_ref.at[pl.ds(my_z * t_loc, t_loc), :],
            send_sem=sem_xs, recv_sem=sem_xr,
            device_id=peer, device_id_type=pl.DeviceIdType.MESH)
        rdma_x.start()
        rdma_r = pltpu.make_async_remote_copy(
            src_ref=r_ref, dst_ref=rin_ref,
            send_sem=sem_rs, recv_sem=sem_rr,
            device_id=peer, device_id_type=pl.DeviceIdType.MESH)
        rdma_r.start()

        xfull_ref[pl.ds(my_z * t_loc, t_loc), :] = x_ref[...]

        rdma_r.wait()
        rdma_x.wait()

        r_my = r_ref[...]
        r_in = rin_ref[...]
        rfull = jnp.where(my_z == 0,
                          jnp.concatenate([r_my, r_in], axis=1),
                          jnp.concatenate([r_in, r_my], axis=1))
        xf = xfull_ref[...]
        g = jnp.dot(xf, rfull, preferred_element_type=jnp.float32)
        m1 = jnp.max(g, axis=1, keepdims=True)
        is1 = g >= m1
        g2 = jnp.where(is1, -jnp.inf, g)
        m2 = jnp.max(g2, axis=1, keepdims=True)
        is2 = g2 >= m2
        e2 = jnp.exp(m2 - m1)
        wfull = (is1.astype(jnp.float32) + is2.astype(jnp.float32) * e2) \
            / (1.0 + e2)
        wloc_ref[...] = lax.dynamic_slice(wfull, (0, my_z * e_loc), (t, e_loc))

        for e in range(e_loc):
            cp1 = pltpu.make_async_copy(w1_any.at[e], w1f_ref, sem_w1)
            cp1.start()
            cp2 = pltpu.make_async_copy(w2_any.at[e], w2f_ref, sem_w2)
            cp2.start()
            cp1.wait()
            cp2.wait()
            w1b_ref[...] = w1f_ref[...].astype(jnp.bfloat16)
            w2b_ref[...] = w2f_ref[...].astype(jnp.bfloat16)

            def tile_body(i, _, e=e):
                rows = pl.ds(i * T_TILE, T_TILE)
                xt = xfull_ref[rows, :].astype(jnp.bfloat16)
                h = jnp.dot(xt, w1b_ref[...],
                            preferred_element_type=jnp.float32)
                h = jnp.maximum(h, 0.0).astype(jnp.bfloat16)
                y = jnp.dot(h, w2b_ref[...],
                            preferred_element_type=jnp.float32)
                contrib = y * wloc_ref[rows, e:e + 1]
                if e == 0:
                    acc_ref[rows, :] = contrib
                else:
                    acc_ref[rows, :] += contrib
                return 0

            lax.fori_loop(0, n_tiles, tile_body, 0)

        sendbuf_ref[...] = acc_ref[pl.ds((1 - my_z) * t_loc, t_loc), :]
        rdma_c = pltpu.make_async_remote_copy(
            src_ref=sendbuf_ref, dst_ref=comb_ref,
            send_sem=sem_cs, recv_sem=sem_cr,
            device_id=peer, device_id_type=pl.DeviceIdType.MESH)
        rdma_c.start()
        rdma_c.wait()
        out_ref[...] = acc_ref[pl.ds(my_z * t_loc, t_loc), :] + comb_ref[...]

    return pl.pallas_call(
        body,
        out_shape=jax.ShapeDtypeStruct((t_loc, d), jnp.float32),
        in_specs=[
            pl.BlockSpec(memory_space=pltpu.VMEM),
            pl.BlockSpec(memory_space=pltpu.VMEM),
            pl.BlockSpec(memory_space=pltpu.ANY),
            pl.BlockSpec(memory_space=pltpu.ANY),
        ],
        out_specs=pl.BlockSpec(memory_space=pltpu.VMEM),
        scratch_shapes=[
            pltpu.VMEM((t, d), jnp.float32),
            pltpu.VMEM((d, e_loc), jnp.float32),
            pltpu.VMEM((t, e_loc), jnp.float32),
            pltpu.VMEM((d, f), jnp.float32),
            pltpu.VMEM((f, d), jnp.float32),
            pltpu.VMEM((d, f), jnp.bfloat16),
            pltpu.VMEM((f, d), jnp.bfloat16),
            pltpu.VMEM((t, d), jnp.float32),
            pltpu.VMEM((t_loc, d), jnp.float32),
            pltpu.VMEM((t_loc, d), jnp.float32),
            pltpu.SemaphoreType.DMA,
            pltpu.SemaphoreType.DMA,
            pltpu.SemaphoreType.DMA,
            pltpu.SemaphoreType.DMA,
            pltpu.SemaphoreType.DMA,
            pltpu.SemaphoreType.DMA,
            pltpu.SemaphoreType.DMA,
            pltpu.SemaphoreType.DMA,
        ],
        compiler_params=pltpu.CompilerParams(collective_id=0),
    )(x, router, W1, W2)


# baseline (device time: 578807 ns/iter reference)
import jax
import jax.numpy as jnp
from jax import lax
from jax.experimental import pallas as pl
from jax.experimental.pallas import tpu as pltpu

N_Z = 2
T_TILE = 256


def kernel(x, router, W1, W2):
    t_loc, d = x.shape
    e_loc, _, f = W1.shape
    t = N_Z * t_loc
    n_tiles = t // T_TILE

    def body(x_ref, r_ref, w1_any, w2_any, out_ref,
             xfull_ref, rin_ref, wloc_ref,
             wchunk_ref, w1b_ref, w2b_ref, acc_ref,
             sem_xs, sem_xr, sem_rs, sem_rr, sem_cs, sem_cr,
             sem_w):
        my_x = lax.axis_index("x")
        my_y = lax.axis_index("y")
        my_z = lax.axis_index("z")
        peer = (my_x, my_y, 1 - my_z)

        barrier_sem = pltpu.get_barrier_semaphore()
        pl.semaphore_signal(barrier_sem, inc=1, device_id=peer,
                            device_id_type=pl.DeviceIdType.MESH)
        pl.semaphore_wait(barrier_sem, 1)

        rdma_x = pltpu.make_async_remote_copy(
            src_ref=x_ref,
            dst_ref=xfull_ref.at[pl.ds(my_z * t_loc, t_loc), :],
            send_sem=sem_xs, recv_sem=sem_xr,
            device_id=peer, device_id_type=pl.DeviceIdType.MESH)
        rdma_x.start()
        rdma_r = pltpu.make_async_remote_copy(
            src_ref=r_ref, dst_ref=rin_ref,
            send_sem=sem_rs, recv_sem=sem_rr,
            device_id=peer, device_id_type=pl.DeviceIdType.MESH)
        rdma_r.start()

        xfull_ref[pl.ds(my_z * t_loc, t_loc), :] = x_ref[...]

        rdma_r.wait()
        rdma_x.wait()

        r_my = r_ref[...]
        r_in = rin_ref[...]
        rfull = jnp.where(my_z == 0,
                          jnp.concatenate([r_my, r_in], axis=1),
                          jnp.concatenate([r_in, r_my], axis=1))
        xf = xfull_ref[...]
        g = jnp.dot(xf, rfull, preferred_element_type=jnp.float32,
                    precision=lax.Precision.HIGHEST)
        m1 = jnp.max(g, axis=1, keepdims=True)
        is1 = g >= m1
        g2 = jnp.where(is1, -jnp.inf, g)
        m2 = jnp.max(g2, axis=1, keepdims=True)
        is2 = g2 >= m2
        e2 = jnp.exp(m2 - m1)
        wfull = (is1.astype(jnp.float32) + is2.astype(jnp.float32) * e2) \
            / (1.0 + e2)
        wloc_ref[...] = jnp.where(my_z == 0,
                                  wfull[:, :e_loc], wfull[:, e_loc:])

        n_chunks = f // d
        for e in range(e_loc):
            for c in range(n_chunks):
                cp = pltpu.make_async_copy(
                    w1_any.at[e, :, pl.ds(c * d, d)], wchunk_ref, sem_w)
                cp.start()
                cp.wait()
                w1b_ref[:, pl.ds(c * d, d)] = \
                    wchunk_ref[...].astype(jnp.bfloat16)
            for c in range(n_chunks):
                cp = pltpu.make_async_copy(
                    w2_any.at[e, pl.ds(c * d, d), :], wchunk_ref, sem_w)
                cp.start()
                cp.wait()
                w2b_ref[pl.ds(c * d, d), :] = \
                    wchunk_ref[...].astype(jnp.bfloat16)

            def tile_body(i, _, e=e):
                rows = pl.ds(i * T_TILE, T_TILE)
                xt = xfull_ref[rows, :].astype(jnp.bfloat16)
                h = jnp.dot(xt, w1b_ref[...],
                            preferred_element_type=jnp.float32)
                h = jnp.maximum(h, 0.0).astype(jnp.bfloat16)
                y = jnp.dot(h, w2b_ref[...],
                            preferred_element_type=jnp.float32)
                contrib = y * wloc_ref[rows, e:e + 1]
                if e == 0:
                    acc_ref[rows, :] = contrib
                else:
                    acc_ref[rows, :] += contrib
                return 0

            lax.fori_loop(0, n_tiles, tile_body, 0)

        rdma_c = pltpu.make_async_remote_copy(
            src_ref=acc_ref.at[pl.ds((1 - my_z) * t_loc, t_loc), :],
            dst_ref=out_ref,
            send_sem=sem_cs, recv_sem=sem_cr,
            device_id=peer, device_id_type=pl.DeviceIdType.MESH)
        rdma_c.start()
        rdma_c.wait()
        out_ref[...] += acc_ref[pl.ds(my_z * t_loc, t_loc), :]

    return pl.pallas_call(
        body,
        out_shape=jax.ShapeDtypeStruct((t_loc, d), jnp.float32),
        in_specs=[
            pl.BlockSpec(memory_space=pltpu.VMEM),
            pl.BlockSpec(memory_space=pltpu.VMEM),
            pl.BlockSpec(memory_space=pl.ANY),
            pl.BlockSpec(memory_space=pl.ANY),
        ],
        out_specs=pl.BlockSpec(memory_space=pltpu.VMEM),
        scratch_shapes=[
            pltpu.VMEM((t, d), jnp.float32),
            pltpu.VMEM((d, e_loc), jnp.float32),
            pltpu.VMEM((t, e_loc), jnp.float32),
            pltpu.VMEM((d, d), jnp.float32),
            pltpu.VMEM((d, f), jnp.bfloat16),
            pltpu.VMEM((f, d), jnp.bfloat16),
            pltpu.VMEM((t, d), jnp.float32),
            pltpu.SemaphoreType.DMA,
            pltpu.SemaphoreType.DMA,
            pltpu.SemaphoreType.DMA,
            pltpu.SemaphoreType.DMA,
            pltpu.SemaphoreType.DMA,
            pltpu.SemaphoreType.DMA,
            pltpu.SemaphoreType.DMA,
        ],
        compiler_params=pltpu.CompilerParams(
            collective_id=0,
            vmem_limit_bytes=100 * 1024 * 1024,
        ),
    )(x, router, W1, W2)
